# baseline (speedup 1.0000x reference)
import functools

import jax
import jax.numpy as jnp
from jax.experimental import pallas as pl
from jax.experimental.pallas import tpu as pltpu

NUM_EXPERTS = 64
TOP_K = 8
HIDDEN = 4096
BLOCK_M = 1024
HALF_M = BLOCK_M // 2


def _topk_from_logits(logits, logits_ref, weights_ref, indices_ref):
    logits_ref[...] = logits

    m = jnp.max(logits, axis=-1, keepdims=True)
    e = jnp.exp(logits - m)
    probs = e / jnp.sum(e, axis=-1, keepdims=True)

    col = jax.lax.broadcasted_iota(jnp.int32, probs.shape, 1)
    work = probs
    vals = []
    idxs = []
    for _ in range(TOP_K):
        mj = jnp.max(work, axis=-1, keepdims=True)
        ij = jnp.min(jnp.where(work == mj, col, NUM_EXPERTS), axis=-1,
                     keepdims=True)
        vals.append(mj)
        idxs.append(ij)
        work = jnp.where(col == ij, -1.0, work)

    top_vals = jnp.concatenate(vals, axis=-1)
    weights_ref[...] = top_vals / jnp.sum(top_vals, axis=-1, keepdims=True)
    indices_ref[...] = jnp.concatenate(idxs, axis=-1)


def _router_block(xa_ref, xb_ref, w_ref, logits_ref, weights_ref,
                  indices_ref):
    w = w_ref[...]
    la = jnp.dot(xa_ref[...], w, preferred_element_type=jnp.float32)
    lb = jnp.dot(xb_ref[...], w, preferred_element_type=jnp.float32)
    _topk_from_logits(jnp.concatenate([la, lb], axis=0), logits_ref,
                      weights_ref, indices_ref)


@jax.jit
def kernel(hidden_states, weight):
    x = hidden_states.reshape(-1, HIDDEN)
    rows = x.shape[0]
    wt = weight.T
    grid = (rows // BLOCK_M,)
    logits, weights, indices = pl.pallas_call(
        _router_block,
        grid=grid,
        in_specs=[
            pl.BlockSpec((HALF_M, HIDDEN), lambda i: (2 * i, 0)),
            pl.BlockSpec((HALF_M, HIDDEN), lambda i: (2 * i + 1, 0)),
            pl.BlockSpec((HIDDEN, NUM_EXPERTS), lambda i: (0, 0)),
        ],
        out_specs=[
            pl.BlockSpec((BLOCK_M, NUM_EXPERTS), lambda i: (i, 0)),
            pl.BlockSpec((BLOCK_M, TOP_K), lambda i: (i, 0)),
            pl.BlockSpec((BLOCK_M, TOP_K), lambda i: (i, 0)),
        ],
        out_shape=[
            jax.ShapeDtypeStruct((rows, NUM_EXPERTS), jnp.float32),
            jax.ShapeDtypeStruct((rows, TOP_K), jnp.float32),
            jax.ShapeDtypeStruct((rows, TOP_K), jnp.int32),
        ],
    )(x, x, wt)
    return logits, weights, indices
